# baseline (speedup 1.0000x reference)
import functools

import jax
import jax.numpy as jnp
from jax.experimental import pallas as pl
from jax.experimental.pallas import tpu as pltpu

D = 768
H = 3072
E = 8
K = 2
BLK = 256


def _ffn_block(grp_ref, x_ref, w1_ref, b1_ref, w2_ref, b2_ref, o_ref):
    h = jax.lax.dot_general(
        x_ref[...], w1_ref[0],
        (((1,), (1,)), ((), ())), preferred_element_type=jnp.float32)
    h = jnp.maximum(h + b1_ref[0], 0.0)
    y = jax.lax.dot_general(
        h, w2_ref[0],
        (((1,), (1,)), ((), ())), preferred_element_type=jnp.float32)
    o_ref[...] = y + b2_ref[0]


def kernel(x, W1, b1, W2, b2, Wr, br):
    orig_shape = x.shape
    xf = x.reshape(-1, D)
    T = xf.shape[0]
    A = T * K
    NBLK = A // BLK + E
    NPAD = NBLK * BLK

    logits = xf @ Wr.T + br
    top2v, top2i = jax.lax.top_k(logits, K)
    probs = jax.nn.softmax(top2v, axis=1)

    eid = top2i.reshape(-1).astype(jnp.int32)
    counts = jnp.bincount(eid, length=E)
    cnt_off = jnp.concatenate([jnp.zeros((1,), jnp.int32),
                               jnp.cumsum(counts)[:-1].astype(jnp.int32)])
    blk_counts = (counts + BLK - 1) // BLK
    blk_cum = jnp.cumsum(blk_counts).astype(jnp.int32)
    pad_start = (jnp.concatenate([jnp.zeros((1,), jnp.int32),
                                  blk_cum[:-1]]) * BLK)

    order = jnp.argsort(eid, stable=True)
    sorted_e = eid[order]
    ranks = jnp.arange(A, dtype=jnp.int32) - cnt_off[sorted_e]
    pos_sorted = pad_start[sorted_e] + ranks
    pos_arr = jnp.zeros((A,), jnp.int32).at[order].set(pos_sorted)
    row_tok = jnp.zeros((NPAD,), jnp.int32).at[pos_sorted].set(
        (order // K).astype(jnp.int32))
    grp = jnp.searchsorted(blk_cum, jnp.arange(NBLK, dtype=jnp.int32),
                           side="right").astype(jnp.int32)
    grp = jnp.minimum(grp, E - 1)

    x_sorted = xf[row_tok]

    grid_spec = pltpu.PrefetchScalarGridSpec(
        num_scalar_prefetch=1,
        grid=(NBLK,),
        in_specs=[
            pl.BlockSpec((BLK, D), lambda g, grp: (g, 0)),
            pl.BlockSpec((1, H, D), lambda g, grp: (grp[g], 0, 0)),
            pl.BlockSpec((1, 1, H), lambda g, grp: (grp[g], 0, 0)),
            pl.BlockSpec((1, D, H), lambda g, grp: (grp[g], 0, 0)),
            pl.BlockSpec((1, 1, D), lambda g, grp: (grp[g], 0, 0)),
        ],
        out_specs=pl.BlockSpec((BLK, D), lambda g, grp: (g, 0)),
    )
    y_sorted = pl.pallas_call(
        _ffn_block,
        grid_spec=grid_spec,
        out_shape=jax.ShapeDtypeStruct((NPAD, D), jnp.float32),
    )(grp, x_sorted, W1, b1.reshape(E, 1, H), W2, b2.reshape(E, 1, D))

    y_assign = y_sorted[pos_arr].reshape(T, K, D)
    out = jnp.sum(probs[:, :, None] * y_assign, axis=1)
    return out.reshape(orig_shape)


# speedup vs baseline: 1.6128x; 1.6128x over previous
import functools

import jax
import jax.numpy as jnp
from jax import lax
from jax.experimental import pallas as pl
from jax.experimental.pallas import tpu as pltpu
from jax.experimental.pallas import tpu_sc as plsc

D = 768
H = 3072
E = 8
K = 2
BLK = 256
T = 4096
A = T * K
NBLK = A // BLK + E
NPAD = NBLK * BLK
GRPPAD = 48
NW = 32
TPT = T // NW
NCH = TPT // 16

_MESH = dict(core_axis_name="c", subcore_axis_name="s")
_SC_PARAMS = dict(
    compiler_params=pltpu.CompilerParams(needs_layout_passes=False))


def _router_body(x_ref, wr_ref, br_ref, o_ref):
    o_ref[...] = lax.dot_general(
        wr_ref[...], x_ref[...],
        (((1,), (1,)), ((), ())), preferred_element_type=jnp.float32,
    ) + br_ref[...]


def _ffn_body(grp_ref, x_ref, w1_ref, b1_ref, w2_ref, b2_ref, w_ref, o_ref):
    h = lax.dot_general(
        x_ref[...], w1_ref[0],
        (((1,), (1,)), ((), ())), preferred_element_type=jnp.float32)
    h = jnp.maximum(h + b1_ref[0], 0.0)
    y = lax.dot_general(
        h, w2_ref[0],
        (((1,), (1,)), ((), ())), preferred_element_type=jnp.float32)
    o_ref[...] = (y + b2_ref[0]) * w_ref[...]


def _wid():
    return lax.axis_index("s") * 2 + lax.axis_index("c")


def _route_body(logitsT, i1_o, i2_o, p1_o, p2_o, cnt_o,
                lbuf, i1b, i2b, p1b, p2b, cntb, sem):
    del sem
    wid = _wid()
    tok0 = wid * TPT
    lanes = jnp.arange(16, dtype=jnp.int32)
    NEG = jnp.full((16,), -1e30, jnp.float32)

    for e in range(E):
        pltpu.sync_copy(logitsT.at[e, pl.ds(tok0, TPT)], lbuf.at[e])

    cnt = [jnp.zeros((), jnp.int32) for _ in range(E)]
    for c in range(NCH):
        sl = pl.ds(c * 16, 16)
        l = [lbuf[e, sl] for e in range(E)]
        m1 = l[0]
        i1 = jnp.zeros((16,), jnp.int32)
        for e in range(1, E):
            gt = l[e] > m1
            i1 = jnp.where(gt, e, i1)
            m1 = jnp.where(gt, l[e], m1)
        m2 = NEG
        i2 = jnp.zeros((16,), jnp.int32)
        for e in range(E):
            v = jnp.where(i1 == e, NEG, l[e])
            gt = v > m2
            i2 = jnp.where(gt, e, i2)
            m2 = jnp.where(gt, v, m2)
        pa = 1.0 / (1.0 + jnp.exp(m2 - m1))
        i1b[sl] = i1
        i2b[sl] = i2
        p1b[sl] = pa
        p2b[sl] = 1.0 - pa
        for e in range(E):
            cnt[e] = (cnt[e]
                      + jnp.sum((i1 == e).astype(jnp.int32))
                      + jnp.sum((i2 == e).astype(jnp.int32)))

    acc = jnp.zeros((16,), jnp.int32)
    for e in range(E):
        acc = jnp.where(lanes == e, jnp.zeros((16,), jnp.int32) + cnt[e], acc)
    cntb[...] = acc

    pltpu.sync_copy(i1b, i1_o.at[pl.ds(tok0, TPT)])
    pltpu.sync_copy(i2b, i2_o.at[pl.ds(tok0, TPT)])
    pltpu.sync_copy(p1b, p1_o.at[pl.ds(tok0, TPT)])
    pltpu.sync_copy(p2b, p2_o.at[pl.ds(tok0, TPT)])
    pltpu.sync_copy(cntb, cnt_o.at[wid])


def _dispatch_body(cnts, i1_i, i2_i, p1_i, p2_i, xf,
                   x_sorted, w_row, pos0_o, pos1_o, grp_o,
                   allcnt, i1b, i2b, p1b, p2b, pos0b, pos1b,
                   xbuf, pchunk, grpb, sem):
    wid = _wid()
    tok0 = wid * TPT
    lanes = jnp.arange(16, dtype=jnp.int32)

    pltpu.sync_copy(cnts, allcnt)
    pltpu.sync_copy(i1_i.at[pl.ds(tok0, TPT)], i1b)
    pltpu.sync_copy(i2_i.at[pl.ds(tok0, TPT)], i2b)
    pltpu.sync_copy(p1_i.at[pl.ds(tok0, TPT)], p1b)
    pltpu.sync_copy(p2_i.at[pl.ds(tok0, TPT)], p2b)

    widv = jnp.zeros((16,), jnp.int32) + wid
    tot = jnp.zeros((16,), jnp.int32)
    myoff = jnp.zeros((16,), jnp.int32)
    for w in range(NW):
        cw = allcnt[w, :]
        tot = tot + cw
        before = jnp.full((16,), w, jnp.int32) < widv
        myoff = myoff + jnp.where(before, cw, jnp.zeros((16,), jnp.int32))

    blk = (tot + (BLK - 1)) // BLK
    bcum = jnp.cumsum(blk)
    pstart = (bcum - blk) * BLK

    base = []
    for e in range(E):
        onehot = lanes == e
        s = jnp.sum(jnp.where(onehot, pstart + myoff,
                              jnp.zeros((16,), jnp.int32)))
        base.append(jnp.zeros((16,), jnp.int32) + s)

    for c in range(NCH):
        sl = pl.ds(c * 16, 16)
        for ib, posb in ((i1b, pos0b), (i2b, pos1b)):
            eid = ib[sl]
            pos = jnp.zeros((16,), jnp.int32)
            for e in range(E):
                m = eid == e
                mi = m.astype(jnp.int32)
                rank = jnp.cumsum(mi) - 1
                pos = jnp.where(m, base[e] + rank, pos)
                base[e] = base[e] + jnp.sum(mi)
            posb[sl] = pos

    for c in range(NCH):
        sl = pl.ds(c * 16, 16)
        pltpu.sync_copy(xf.at[pl.ds(tok0 + c * 16, 16)], xbuf)
        pltpu.async_copy(xbuf, x_sorted.at[pos0b[sl]], sem).wait()
        pltpu.async_copy(xbuf, x_sorted.at[pos1b[sl]], sem).wait()
        pchunk[...] = p1b[sl]
        pltpu.async_copy(pchunk, w_row.at[pos0b[sl]], sem).wait()
        pchunk[...] = p2b[sl]
        pltpu.async_copy(pchunk, w_row.at[pos1b[sl]], sem).wait()

    pltpu.sync_copy(pos0b, pos0_o.at[pl.ds(tok0, TPT)])
    pltpu.sync_copy(pos1b, pos1_o.at[pl.ds(tok0, TPT)])

    for q in range(GRPPAD // 16):
        g = jnp.arange(16, dtype=jnp.int32) + (q * 16)
        a = jnp.zeros((16,), jnp.int32)
        for e in range(E):
            s = jnp.sum(jnp.where(lanes == e, bcum,
                                  jnp.zeros((16,), jnp.int32)))
            a = a + jnp.where(jnp.zeros((16,), jnp.int32) + s <= g,
                              jnp.ones((16,), jnp.int32),
                              jnp.zeros((16,), jnp.int32))
        grpb[pl.ds(q * 16, 16)] = jnp.minimum(a, E - 1)

    @pl.when(wid == 0)
    def _():
        pltpu.sync_copy(grpb, grp_o)


def _combine_body(y_sorted, pos0, pos1, out, q0b, q1b, y0, y1, ob, sem):
    wid = _wid()
    tok0 = wid * TPT
    pltpu.sync_copy(pos0.at[pl.ds(tok0, TPT)], q0b)
    pltpu.sync_copy(pos1.at[pl.ds(tok0, TPT)], q1b)

    def chunk(c, carry):
        sl = pl.ds(c * 16, 16)
        pltpu.async_copy(y_sorted.at[q0b[sl]], y0, sem).wait()
        pltpu.async_copy(y_sorted.at[q1b[sl]], y1, sem).wait()
        for j in range(16):
            for cc in range(D // 16):
                csl = pl.ds(cc * 16, 16)
                ob[j, csl] = y0[j, csl] + y1[j, csl]
        pltpu.sync_copy(ob, out.at[pl.ds(tok0 + c * 16, 16)])
        return carry

    lax.fori_loop(0, NCH, chunk, 0)


def _make_route(mesh):
    return functools.partial(
        pl.kernel,
        out_type=[
            jax.ShapeDtypeStruct((T,), jnp.int32),
            jax.ShapeDtypeStruct((T,), jnp.int32),
            jax.ShapeDtypeStruct((T,), jnp.float32),
            jax.ShapeDtypeStruct((T,), jnp.float32),
            jax.ShapeDtypeStruct((NW, 16), jnp.int32),
        ],
        mesh=mesh,
        scratch_types=[
            pltpu.VMEM((E, TPT), jnp.float32),
            pltpu.VMEM((TPT,), jnp.int32),
            pltpu.VMEM((TPT,), jnp.int32),
            pltpu.VMEM((TPT,), jnp.float32),
            pltpu.VMEM((TPT,), jnp.float32),
            pltpu.VMEM((16,), jnp.int32),
            pltpu.SemaphoreType.DMA,
        ],
        **_SC_PARAMS,
    )(_route_body)


def _make_dispatch(mesh):
    return functools.partial(
        pl.kernel,
        out_type=[
            jax.ShapeDtypeStruct((NPAD, D), jnp.float32),
            jax.ShapeDtypeStruct((NPAD,), jnp.float32),
            jax.ShapeDtypeStruct((T,), jnp.int32),
            jax.ShapeDtypeStruct((T,), jnp.int32),
            jax.ShapeDtypeStruct((GRPPAD,), jnp.int32),
        ],
        mesh=mesh,
        scratch_types=[
            pltpu.VMEM((NW, 16), jnp.int32),
            pltpu.VMEM((TPT,), jnp.int32),
            pltpu.VMEM((TPT,), jnp.int32),
            pltpu.VMEM((TPT,), jnp.float32),
            pltpu.VMEM((TPT,), jnp.float32),
            pltpu.VMEM((TPT,), jnp.int32),
            pltpu.VMEM((TPT,), jnp.int32),
            pltpu.VMEM((16, D), jnp.float32),
            pltpu.VMEM((16,), jnp.float32),
            pltpu.VMEM((GRPPAD,), jnp.int32),
            pltpu.SemaphoreType.DMA,
        ],
        **_SC_PARAMS,
    )(_dispatch_body)


def _make_combine(mesh):
    return functools.partial(
        pl.kernel,
        out_type=jax.ShapeDtypeStruct((T, D), jnp.float32),
        mesh=mesh,
        scratch_types=[
            pltpu.VMEM((TPT,), jnp.int32),
            pltpu.VMEM((TPT,), jnp.int32),
            pltpu.VMEM((16, D), jnp.float32),
            pltpu.VMEM((16, D), jnp.float32),
            pltpu.VMEM((16, D), jnp.float32),
            pltpu.SemaphoreType.DMA,
        ],
        **_SC_PARAMS,
    )(_combine_body)


def kernel(x, W1, b1, W2, b2, Wr, br):
    orig_shape = x.shape
    xf = x.reshape(T, D)

    logitsT = pl.pallas_call(
        _router_body,
        out_shape=jax.ShapeDtypeStruct((E, T), jnp.float32),
    )(xf, Wr, br.reshape(E, 1))

    mesh = plsc.VectorSubcoreMesh(**_MESH)

    i1, i2, p1, p2, cnts = _make_route(mesh)(logitsT)

    x_sorted, w_row, pos0, pos1, grp = _make_dispatch(mesh)(
        cnts, i1, i2, p1, p2, xf)

    grid_spec = pltpu.PrefetchScalarGridSpec(
        num_scalar_prefetch=1,
        grid=(NBLK,),
        in_specs=[
            pl.BlockSpec((BLK, D), lambda g, grp: (g, 0)),
            pl.BlockSpec((1, H, D), lambda g, grp: (grp[g], 0, 0)),
            pl.BlockSpec((1, 1, H), lambda g, grp: (grp[g], 0, 0)),
            pl.BlockSpec((1, D, H), lambda g, grp: (grp[g], 0, 0)),
            pl.BlockSpec((1, 1, D), lambda g, grp: (grp[g], 0, 0)),
            pl.BlockSpec((BLK, 1), lambda g, grp: (g, 0)),
        ],
        out_specs=pl.BlockSpec((BLK, D), lambda g, grp: (g, 0)),
    )
    y_sorted = pl.pallas_call(
        _ffn_body,
        grid_spec=grid_spec,
        out_shape=jax.ShapeDtypeStruct((NPAD, D), jnp.float32),
    )(grp[:NBLK], x_sorted, W1, b1.reshape(E, 1, H), W2,
      b2.reshape(E, 1, D), w_row.reshape(NPAD, 1))

    out = _make_combine(mesh)(y_sorted, pos0, pos1)
    return out.reshape(orig_shape)
